# baseline (speedup 1.0000x reference)
import functools

import jax
import jax.numpy as jnp
from jax import lax
from jax.experimental import pallas as pl
from jax.experimental.pallas import tpu as pltpu
from jax.experimental.pallas import tpu_sc as plsc

V = 100000
D = 128
R = 128
TOK = 1024 * 200


_VB = 1000


def _tables_body(a1, b1, a2, b2, a3, b3, o1, o2, o3):
    o1[...] = 2.0 * jnp.dot(a1[...], b1[...], preferred_element_type=jnp.float32)
    o2[...] = 2.0 * jnp.dot(a2[...], b2[...], preferred_element_type=jnp.float32)
    o3[...] = 2.0 * jnp.dot(a3[...], b3[...], preferred_element_type=jnp.float32)


def _make_tables(a1, b1, a2, b2, a3, b3):
    grid = (V // _VB,)
    a_spec = pl.BlockSpec((_VB, R), lambda i: (i, 0))
    b_spec = pl.BlockSpec((R, D), lambda i: (0, 0))
    o_spec = pl.BlockSpec((_VB, D), lambda i: (i, 0))
    out_shape = jax.ShapeDtypeStruct((V, D), jnp.float32)
    return pl.pallas_call(
        _tables_body,
        grid=grid,
        in_specs=[a_spec, b_spec, a_spec, b_spec, a_spec, b_spec],
        out_specs=[o_spec, o_spec, o_spec],
        out_shape=[out_shape, out_shape, out_shape],
    )(a1, b1, a2, b2, a3, b3)


_CH = 128
_NW = 32
_TPW = TOK // _NW
_NCH = _TPW // _CH
_NROWCH = TOK // _CH


def _sc_body(x_hbm, m1_hbm, m2_hbm, m3_hbm, w_hbm, t1_hbm, t2_hbm, t3_hbm,
             out_hbm, xb, i1, i2, i3, r0, r1, r2, r3, sem):
    nc = 2
    wid = lax.axis_index("s") * nc + lax.axis_index("c")
    base = wid * _TPW
    row0 = wid * _NCH

    pltpu.sync_copy(x_hbm.at[pl.ds(base, _TPW)], xb)
    pltpu.sync_copy(m1_hbm.at[pl.ds(base, _TPW)], i1)
    pltpu.sync_copy(m2_hbm.at[pl.ds(base, _TPW)], i2)
    pltpu.sync_copy(m3_hbm.at[pl.ds(base, _TPW)], i3)

    def idx_body(g, _):
        s = pl.ds(g * 16, 16)
        xv = xb[s]
        i1[s] = xv * i1[s]
        i2[s] = xv * i2[s]
        i3[s] = xv * i3[s]
        return 0

    lax.fori_loop(0, _TPW // 16, idx_body, 0)

    def chunk_body(c, _):
        s = pl.ds(c * _CH, _CH)
        d0 = pltpu.async_copy(w_hbm.at[xb.at[s]], r0, sem)
        d1 = pltpu.async_copy(t1_hbm.at[i1.at[s]], r1, sem)
        d2 = pltpu.async_copy(t2_hbm.at[i2.at[s]], r2, sem)
        d3 = pltpu.async_copy(t3_hbm.at[i3.at[s]], r3, sem)
        d0.wait()
        d1.wait()
        d2.wait()
        d3.wait()

        def row_body(r, _):
            for g in range(D // 16):
                s = pl.ds(g * 16, 16)
                r0[r, s] = (r0[r, s] + r1[r, s]) + (r2[r, s] + r3[r, s])
            return 0

        lax.fori_loop(0, _CH, row_body, 0)
        pltpu.sync_copy(r0, out_hbm.at[pl.ds((row0 + c) * _CH, _CH)])
        return 0

    lax.fori_loop(0, _NCH, chunk_body, 0)


def _sc_lookup(x_flat, m1_flat, m2_flat, m3_flat, weight, t1, t2, t3):
    mesh = plsc.VectorSubcoreMesh(core_axis_name="c", subcore_axis_name="s")
    fn = functools.partial(
        pl.kernel,
        mesh=mesh,
        out_type=jax.ShapeDtypeStruct((TOK, D), jnp.float32),
        scratch_types=[
            pltpu.VMEM((_TPW,), jnp.int32),
            pltpu.VMEM((_TPW,), jnp.int32),
            pltpu.VMEM((_TPW,), jnp.int32),
            pltpu.VMEM((_TPW,), jnp.int32),
            pltpu.VMEM((_CH, D), jnp.float32),
            pltpu.VMEM((_CH, D), jnp.float32),
            pltpu.VMEM((_CH, D), jnp.float32),
            pltpu.VMEM((_CH, D), jnp.float32),
            pltpu.SemaphoreType.DMA,
        ],
    )(_sc_body)
    return fn(x_flat, m1_flat, m2_flat, m3_flat, weight, t1, t2, t3)


def kernel(x, cl_mask, lm_mask, cl_prime_mask, weight,
           lora_A_cl, lora_B_cl, lora_A_lm, lora_B_lm,
           lora_A_cl_prime, lora_B_cl_prime):
    b, l = x.shape
    t1, t2, t3 = _make_tables(lora_A_cl, lora_B_cl, lora_A_lm, lora_B_lm,
                              lora_A_cl_prime, lora_B_cl_prime)
    out = _sc_lookup(x.reshape(TOK), cl_mask.reshape(TOK),
                     lm_mask.reshape(TOK), cl_prime_mask.reshape(TOK),
                     weight, t1, t2, t3)
    return out.reshape(b, l, D)


# speedup vs baseline: 2.6402x; 2.6402x over previous
import functools

import jax
import jax.numpy as jnp
from jax import lax
from jax.experimental import pallas as pl
from jax.experimental.pallas import tpu as pltpu
from jax.experimental.pallas import tpu_sc as plsc

V = 100000
D = 128
R = 128
TOK = 1024 * 200


_VB = 1000


def _u_body(w, a1, b1, a2, b2, a3, b3, o):
    o[:, 0:D] = w[...]
    o[:, D:2 * D] = 2.0 * jnp.dot(a1[...], b1[...],
                                  preferred_element_type=jnp.float32)
    o[:, 2 * D:3 * D] = 2.0 * jnp.dot(a2[...], b2[...],
                                      preferred_element_type=jnp.float32)
    o[:, 3 * D:4 * D] = 2.0 * jnp.dot(a3[...], b3[...],
                                      preferred_element_type=jnp.float32)


def _make_table(w, a1, b1, a2, b2, a3, b3):
    grid = (V // _VB,)
    r_spec = pl.BlockSpec((_VB, R), lambda i: (i, 0))
    b_spec = pl.BlockSpec((R, D), lambda i: (0, 0))
    o_spec = pl.BlockSpec((_VB, 4 * D), lambda i: (i, 0))
    return pl.pallas_call(
        _u_body,
        grid=grid,
        in_specs=[r_spec, r_spec, b_spec, r_spec, b_spec, r_spec, b_spec],
        out_specs=o_spec,
        out_shape=jax.ShapeDtypeStruct((V, 4 * D), jnp.float32),
    )(w, a1, b1, a2, b2, a3, b3)


def _bcast_lane(vec, lane):
    idx = jnp.full((16, 1), lane, jnp.int32)
    dn = lax.GatherDimensionNumbers(
        offset_dims=(), collapsed_slice_dims=(0,), start_index_map=(0,))
    return lax.gather(vec, idx, dn, (1,),
                      mode=lax.GatherScatterMode.PROMISE_IN_BOUNDS)


_CH = 64
_NW = 32
_TPW = TOK // _NW
_NCH = _TPW // _CH


def _sc_body(x_hbm, m1_hbm, m2_hbm, m3_hbm, u_hbm, out_hbm,
             xb, p1, p2, p3, cb, c0b, rb0, rb1, ob0, ob1,
             g0, g1, o0, o1):
    nc = 2
    wid = lax.axis_index("s") * nc + lax.axis_index("c")
    base = wid * _TPW

    pltpu.sync_copy(x_hbm.at[pl.ds(base, _TPW)], xb)
    pltpu.sync_copy(m1_hbm.at[pl.ds(base, _TPW)], p1)
    pltpu.sync_copy(m2_hbm.at[pl.ds(base, _TPW)], p2)
    pltpu.sync_copy(m3_hbm.at[pl.ds(base, _TPW)], p3)
    pltpu.sync_copy(u_hbm.at[pl.ds(0, 1)], cb)

    def idx_body(g, _):
        s = pl.ds(g * 16, 16)
        xv = xb[s]
        p1[s] = xv * p1[s]
        p2[s] = xv * p2[s]
        p3[s] = xv * p3[s]
        return 0

    lax.fori_loop(0, _TPW // 16, idx_body, 0)

    for g in range(D // 16):
        s = pl.ds(g * 16, 16)
        c0b[0, s] = (cb[0, pl.ds(D + g * 16, 16)]
                     + cb[0, pl.ds(2 * D + g * 16, 16)]
                     + cb[0, pl.ds(3 * D + g * 16, 16)])

    def fire_gather(c, rb, sem):
        return pltpu.async_copy(u_hbm.at[xb.at[pl.ds(c * _CH, _CH)]], rb, sem)

    def wait_gather(rb, sem):
        pltpu.make_async_copy(u_hbm.at[pl.ds(0, _CH)], rb, sem).wait()

    def wait_out(ob, sem):
        pltpu.make_async_copy(ob, out_hbm.at[pl.ds(0, _CH)], sem).wait()

    def combine(rb, ob, c):
        toff0 = c * _CH

        def grp_body(q, _):
            gsl = pl.ds(toff0 + q * 16, 16)
            one = jnp.full((16,), 1, jnp.int32)
            iv1 = jnp.minimum(p1[gsl], one).astype(jnp.float32)
            iv2 = jnp.minimum(p2[gsl], one).astype(jnp.float32)
            iv3 = jnp.minimum(p3[gsl], one).astype(jnp.float32)
            for rr in range(16):
                r = q * 16 + rr
                b1 = _bcast_lane(iv1, rr)
                b2 = _bcast_lane(iv2, rr)
                b3 = _bcast_lane(iv3, rr)
                for g in range(D // 16):
                    s = pl.ds(g * 16, 16)
                    s1 = pl.ds(D + g * 16, 16)
                    s2 = pl.ds(2 * D + g * 16, 16)
                    s3 = pl.ds(3 * D + g * 16, 16)
                    acc = rb[r, s] + c0b[0, s]
                    acc = acc + b1 * (rb[r, s1] - cb[0, s1])
                    acc = acc + b2 * (rb[r, s2] - cb[0, s2])
                    acc = acc + b3 * (rb[r, s3] - cb[0, s3])
                    ob[r, s] = acc
            return 0

        lax.fori_loop(0, _CH // 16, grp_body, 0)

    fire_gather(0, rb0, g0)
    fire_gather(1, rb1, g1)

    def step_wrap(j, c):
        wait_gather(rb0, g0)

        @pl.when(j > 0)
        def _():
            wait_out(ob0, o0)

        combine(rb0, ob0, c)
        pltpu.async_copy(ob0, out_hbm.at[pl.ds(base + c * _CH, _CH)], o0)

        @pl.when(c + 2 < _NCH)
        def _():
            fire_gather(c + 2, rb0, g0)

        wait_gather(rb1, g1)

        @pl.when(j > 0)
        def _():
            wait_out(ob1, o1)

        combine(rb1, ob1, c + 1)
        pltpu.async_copy(ob1, out_hbm.at[pl.ds(base + (c + 1) * _CH, _CH)], o1)

        @pl.when(c + 3 < _NCH)
        def _():
            fire_gather(c + 3, rb1, g1)

        return c + 2

    lax.fori_loop(0, _NCH // 2, step_wrap, 0)
    wait_out(ob0, o0)
    wait_out(ob1, o1)


def _sc_lookup(x_flat, m1_flat, m2_flat, m3_flat, u):
    mesh = plsc.VectorSubcoreMesh(core_axis_name="c", subcore_axis_name="s")
    fn = functools.partial(
        pl.kernel,
        mesh=mesh,
        out_type=jax.ShapeDtypeStruct((TOK, D), jnp.float32),
        scratch_types=[
            pltpu.VMEM((_TPW,), jnp.int32),
            pltpu.VMEM((_TPW,), jnp.int32),
            pltpu.VMEM((_TPW,), jnp.int32),
            pltpu.VMEM((_TPW,), jnp.int32),
            pltpu.VMEM((1, 4 * D), jnp.float32),
            pltpu.VMEM((1, D), jnp.float32),
            pltpu.VMEM((_CH, 4 * D), jnp.float32),
            pltpu.VMEM((_CH, 4 * D), jnp.float32),
            pltpu.VMEM((_CH, D), jnp.float32),
            pltpu.VMEM((_CH, D), jnp.float32),
            pltpu.SemaphoreType.DMA,
            pltpu.SemaphoreType.DMA,
            pltpu.SemaphoreType.DMA,
            pltpu.SemaphoreType.DMA,
        ],
    )(_sc_body)
    return fn(x_flat, m1_flat, m2_flat, m3_flat, u)


def kernel(x, cl_mask, lm_mask, cl_prime_mask, weight,
           lora_A_cl, lora_B_cl, lora_A_lm, lora_B_lm,
           lora_A_cl_prime, lora_B_cl_prime):
    b, l = x.shape
    u = _make_table(weight, lora_A_cl, lora_B_cl, lora_A_lm, lora_B_lm,
                    lora_A_cl_prime, lora_B_cl_prime)
    out = _sc_lookup(x.reshape(TOK), cl_mask.reshape(TOK),
                     lm_mask.reshape(TOK), cl_prime_mask.reshape(TOK), u)
    return out.reshape(b, l, D)


# speedup vs baseline: 6.8225x; 2.5841x over previous
import functools

import jax
import jax.numpy as jnp
from jax import lax
from jax.experimental import pallas as pl
from jax.experimental.pallas import tpu as pltpu
from jax.experimental.pallas import tpu_sc as plsc

V = 100000
D = 128
R = 128
TOK = 1024 * 200


_VB = 1000


def _u_body(w, a1, b1, a2, b2, a3, b3, a01, a02, a03, o):
    t01 = 2.0 * jnp.dot(a01[...], b1[...], preferred_element_type=jnp.float32)
    t02 = 2.0 * jnp.dot(a02[...], b2[...], preferred_element_type=jnp.float32)
    t03 = 2.0 * jnp.dot(a03[...], b3[...], preferred_element_type=jnp.float32)
    o[:, 0:D] = w[...] + (t01 + (t02 + t03))
    o[:, D:2 * D] = 2.0 * jnp.dot(a1[...], b1[...],
                                  preferred_element_type=jnp.float32) - t01
    o[:, 2 * D:3 * D] = 2.0 * jnp.dot(a2[...], b2[...],
                                      preferred_element_type=jnp.float32) - t02
    o[:, 3 * D:4 * D] = 2.0 * jnp.dot(a3[...], b3[...],
                                      preferred_element_type=jnp.float32) - t03


def _make_table(w, a1, b1, a2, b2, a3, b3):
    grid = (V // _VB,)
    r_spec = pl.BlockSpec((_VB, R), lambda i: (i, 0))
    b_spec = pl.BlockSpec((R, D), lambda i: (0, 0))
    r0_spec = pl.BlockSpec((1, R), lambda i: (0, 0))
    o_spec = pl.BlockSpec((_VB, 4 * D), lambda i: (i, 0))
    return pl.pallas_call(
        _u_body,
        grid=grid,
        in_specs=[r_spec, r_spec, b_spec, r_spec, b_spec, r_spec, b_spec,
                  r0_spec, r0_spec, r0_spec],
        out_specs=o_spec,
        out_shape=jax.ShapeDtypeStruct((V, 4 * D), jnp.float32),
    )(w, a1, b1, a2, b2, a3, b3, a1[0:1], a2[0:1], a3[0:1])


def _bcast_lane(vec, lane):
    idx = jnp.full((16, 1), lane, jnp.int32)
    dn = lax.GatherDimensionNumbers(
        offset_dims=(), collapsed_slice_dims=(0,), start_index_map=(0,))
    return lax.gather(vec, idx, dn, (1,),
                      mode=lax.GatherScatterMode.PROMISE_IN_BOUNDS)


_CH = 64
_NW = 32
_TPW = TOK // _NW
_NCH = _TPW // _CH


def _sc_body(x_hbm, m1_hbm, m2_hbm, m3_hbm, u_hbm, out_hbm,
             xb, p1, p2, p3, rb0, rb1, ob0, ob1,
             g0, g1, o0, o1):
    nc = 2
    wid = lax.axis_index("s") * nc + lax.axis_index("c")
    base = wid * _TPW

    pltpu.sync_copy(x_hbm.at[pl.ds(base, _TPW)], xb)
    pltpu.sync_copy(m1_hbm.at[pl.ds(base, _TPW)], p1)
    pltpu.sync_copy(m2_hbm.at[pl.ds(base, _TPW)], p2)
    pltpu.sync_copy(m3_hbm.at[pl.ds(base, _TPW)], p3)

    def idx_body(g, _):
        s = pl.ds(g * 16, 16)
        xv = xb[s]
        p1[s] = xv * p1[s]
        p2[s] = xv * p2[s]
        p3[s] = xv * p3[s]
        return 0

    lax.fori_loop(0, _TPW // 16, idx_body, 0)

    def fire_gather(c, rb, sem):
        return pltpu.async_copy(u_hbm.at[xb.at[pl.ds(c * _CH, _CH)]], rb, sem)

    def wait_gather(rb, sem):
        pltpu.make_async_copy(u_hbm.at[pl.ds(0, _CH)], rb, sem).wait()

    def wait_out(ob, sem):
        pltpu.make_async_copy(ob, out_hbm.at[pl.ds(0, _CH)], sem).wait()

    def combine(rb, ob, c):
        toff0 = c * _CH

        def grp_body(q, _):
            gsl = pl.ds(toff0 + q * 16, 16)
            one = jnp.full((16,), 1, jnp.int32)
            iv1 = jnp.minimum(p1[gsl], one).astype(jnp.float32)
            iv2 = jnp.minimum(p2[gsl], one).astype(jnp.float32)
            iv3 = jnp.minimum(p3[gsl], one).astype(jnp.float32)
            for rr in range(16):
                r = q * 16 + rr
                b1 = _bcast_lane(iv1, rr)
                b2 = _bcast_lane(iv2, rr)
                b3 = _bcast_lane(iv3, rr)
                for g in range(D // 16):
                    s = pl.ds(g * 16, 16)
                    s1 = pl.ds(D + g * 16, 16)
                    s2 = pl.ds(2 * D + g * 16, 16)
                    s3 = pl.ds(3 * D + g * 16, 16)
                    ob[r, s] = ((rb[r, s] + b1 * rb[r, s1])
                                + (b2 * rb[r, s2] + b3 * rb[r, s3]))
            return 0

        lax.fori_loop(0, _CH // 16, grp_body, 0)

    fire_gather(0, rb0, g0)
    fire_gather(1, rb1, g1)

    def step_wrap(j, c):
        wait_gather(rb0, g0)

        @pl.when(j > 0)
        def _():
            wait_out(ob0, o0)

        combine(rb0, ob0, c)
        pltpu.async_copy(ob0, out_hbm.at[pl.ds(base + c * _CH, _CH)], o0)

        @pl.when(c + 2 < _NCH)
        def _():
            fire_gather(c + 2, rb0, g0)

        wait_gather(rb1, g1)

        @pl.when(j > 0)
        def _():
            wait_out(ob1, o1)

        combine(rb1, ob1, c + 1)
        pltpu.async_copy(ob1, out_hbm.at[pl.ds(base + (c + 1) * _CH, _CH)], o1)

        @pl.when(c + 3 < _NCH)
        def _():
            fire_gather(c + 3, rb1, g1)

        return c + 2

    lax.fori_loop(0, _NCH // 2, step_wrap, 0)
    wait_out(ob0, o0)
    wait_out(ob1, o1)


def _sc_lookup(x_flat, m1_flat, m2_flat, m3_flat, u):
    mesh = plsc.VectorSubcoreMesh(core_axis_name="c", subcore_axis_name="s")
    fn = functools.partial(
        pl.kernel,
        mesh=mesh,
        out_type=jax.ShapeDtypeStruct((TOK, D), jnp.float32),
        scratch_types=[
            pltpu.VMEM((_TPW,), jnp.int32),
            pltpu.VMEM((_TPW,), jnp.int32),
            pltpu.VMEM((_TPW,), jnp.int32),
            pltpu.VMEM((_TPW,), jnp.int32),
            pltpu.VMEM((_CH, 4 * D), jnp.float32),
            pltpu.VMEM((_CH, 4 * D), jnp.float32),
            pltpu.VMEM((_CH, D), jnp.float32),
            pltpu.VMEM((_CH, D), jnp.float32),
            pltpu.SemaphoreType.DMA,
            pltpu.SemaphoreType.DMA,
            pltpu.SemaphoreType.DMA,
            pltpu.SemaphoreType.DMA,
        ],
    )(_sc_body)
    return fn(x_flat, m1_flat, m2_flat, m3_flat, u)


def kernel(x, cl_mask, lm_mask, cl_prime_mask, weight,
           lora_A_cl, lora_B_cl, lora_A_lm, lora_B_lm,
           lora_A_cl_prime, lora_B_cl_prime):
    b, l = x.shape
    u = _make_table(weight, lora_A_cl, lora_B_cl, lora_A_lm, lora_B_lm,
                    lora_A_cl_prime, lora_B_cl_prime)
    out = _sc_lookup(x.reshape(TOK), cl_mask.reshape(TOK),
                     lm_mask.reshape(TOK), cl_prime_mask.reshape(TOK), u)
    return out.reshape(b, l, D)


# speedup vs baseline: 9.2295x; 1.3528x over previous
import functools

import numpy as np

import jax
import jax.numpy as jnp
from jax import lax
from jax.experimental import pallas as pl
from jax.experimental.pallas import tpu as pltpu
from jax.experimental.pallas import tpu_sc as plsc

V = 100000
D = 128
R = 128
TOK = 1024 * 200

_PK = 4 * D // 2


_VB = 1000


def _pack_section(o, i, s):
    u16v = lax.bitcast_convert_type(s.astype(jnp.bfloat16), jnp.uint16)
    acols = jnp.concatenate(
        [u16v[:, 0:16], u16v[:, 32:48], u16v[:, 64:80], u16v[:, 96:112]],
        axis=1).astype(jnp.uint32)
    bcols = jnp.concatenate(
        [u16v[:, 16:32], u16v[:, 48:64], u16v[:, 80:96], u16v[:, 112:128]],
        axis=1).astype(jnp.uint32)
    o[:, 64 * i:64 * i + 64] = acols | (bcols << 16)


def _u_body(w, a1, b1, a2, b2, a3, b3, a01, a02, a03, o):
    t01 = 2.0 * jnp.dot(a01[...], b1[...], preferred_element_type=jnp.float32)
    t02 = 2.0 * jnp.dot(a02[...], b2[...], preferred_element_type=jnp.float32)
    t03 = 2.0 * jnp.dot(a03[...], b3[...], preferred_element_type=jnp.float32)
    _pack_section(o, 0, w[...] + (t01 + (t02 + t03)))
    _pack_section(o, 1, 2.0 * jnp.dot(a1[...], b1[...],
                                      preferred_element_type=jnp.float32) - t01)
    _pack_section(o, 2, 2.0 * jnp.dot(a2[...], b2[...],
                                      preferred_element_type=jnp.float32) - t02)
    _pack_section(o, 3, 2.0 * jnp.dot(a3[...], b3[...],
                                      preferred_element_type=jnp.float32) - t03)


def _make_table(w, a1, b1, a2, b2, a3, b3):
    grid = (V // _VB,)
    r_spec = pl.BlockSpec((_VB, R), lambda i: (i, 0))
    b_spec = pl.BlockSpec((R, D), lambda i: (0, 0))
    r0_spec = pl.BlockSpec((1, R), lambda i: (0, 0))
    o_spec = pl.BlockSpec((_VB, _PK), lambda i: (i, 0))
    return pl.pallas_call(
        _u_body,
        grid=grid,
        in_specs=[r_spec, r_spec, b_spec, r_spec, b_spec, r_spec, b_spec,
                  r0_spec, r0_spec, r0_spec],
        out_specs=o_spec,
        out_shape=jax.ShapeDtypeStruct((V, _PK), jnp.uint32),
    )(w, a1, b1, a2, b2, a3, b3, a1[0:1], a2[0:1], a3[0:1])


def _bcast_lane(vec, lane):
    idx = jnp.full((16, 1), lane, jnp.int32)
    dn = lax.GatherDimensionNumbers(
        offset_dims=(), collapsed_slice_dims=(0,), start_index_map=(0,))
    return lax.gather(vec, idx, dn, (1,),
                      mode=lax.GatherScatterMode.PROMISE_IN_BOUNDS)


_CH = 128
_NW = 32
_TPW = TOK // _NW
_NCH = _TPW // _CH

def _unpack2(v):
    a = lax.bitcast_convert_type(v << 16, jnp.float32)
    b = lax.bitcast_convert_type(v & np.uint32(0xFFFF0000), jnp.float32)
    return a, b


def _sc_body(x_hbm, m1_hbm, m2_hbm, m3_hbm, u_hbm, out_hbm,
             xb, p1, p2, p3, rb0, rb1, ob0, ob1,
             g0, g1, o0, o1):
    nc = 2
    wid = lax.axis_index("s") * nc + lax.axis_index("c")
    base = wid * _TPW

    pltpu.sync_copy(x_hbm.at[pl.ds(base, _TPW)], xb)
    pltpu.sync_copy(m1_hbm.at[pl.ds(base, _TPW)], p1)
    pltpu.sync_copy(m2_hbm.at[pl.ds(base, _TPW)], p2)
    pltpu.sync_copy(m3_hbm.at[pl.ds(base, _TPW)], p3)

    def idx_body(g, _):
        s = pl.ds(g * 16, 16)
        xv = xb[s]
        p1[s] = xv * p1[s]
        p2[s] = xv * p2[s]
        p3[s] = xv * p3[s]
        return 0

    lax.fori_loop(0, _TPW // 16, idx_body, 0)

    def fire_gather(c, rb, sem):
        return pltpu.async_copy(u_hbm.at[xb.at[pl.ds(c * _CH, _CH)]], rb, sem)

    def wait_gather(rb, sem):
        pltpu.make_async_copy(u_hbm.at[pl.ds(0, _CH)], rb, sem).wait()

    def wait_out(ob, sem):
        pltpu.make_async_copy(ob, out_hbm.at[pl.ds(0, _CH)], sem).wait()

    def combine(rb, ob, c):
        toff0 = c * _CH

        def grp_body(q, _):
            gsl = pl.ds(toff0 + q * 16, 16)
            one = jnp.full((16,), 1, jnp.int32)
            iv1 = jnp.minimum(p1[gsl], one).astype(jnp.float32)
            iv2 = jnp.minimum(p2[gsl], one).astype(jnp.float32)
            iv3 = jnp.minimum(p3[gsl], one).astype(jnp.float32)
            for rr in range(16):
                r = q * 16 + rr
                b1 = _bcast_lane(iv1, rr)
                b2 = _bcast_lane(iv2, rr)
                b3 = _bcast_lane(iv3, rr)
                for g2 in range(4):
                    wv = rb[r, pl.ds(g2 * 16, 16)]
                    t1 = rb[r, pl.ds(64 + g2 * 16, 16)]
                    t2 = rb[r, pl.ds(128 + g2 * 16, 16)]
                    t3 = rb[r, pl.ds(192 + g2 * 16, 16)]
                    wa, wb = _unpack2(wv)
                    t1a, t1b = _unpack2(t1)
                    t2a, t2b = _unpack2(t2)
                    t3a, t3b = _unpack2(t3)
                    ob[r, pl.ds(32 * g2, 16)] = (
                        (wa + b1 * t1a) + (b2 * t2a + b3 * t3a))
                    ob[r, pl.ds(32 * g2 + 16, 16)] = (
                        (wb + b1 * t1b) + (b2 * t2b + b3 * t3b))
            return 0

        lax.fori_loop(0, _CH // 16, grp_body, 0)

    fire_gather(0, rb0, g0)
    fire_gather(1, rb1, g1)

    def step_wrap(j, c):
        wait_gather(rb0, g0)

        @pl.when(j > 0)
        def _():
            wait_out(ob0, o0)

        combine(rb0, ob0, c)
        pltpu.async_copy(ob0, out_hbm.at[pl.ds(base + c * _CH, _CH)], o0)

        @pl.when(c + 2 < _NCH)
        def _():
            fire_gather(c + 2, rb0, g0)

        wait_gather(rb1, g1)

        @pl.when(j > 0)
        def _():
            wait_out(ob1, o1)

        combine(rb1, ob1, c + 1)
        pltpu.async_copy(ob1, out_hbm.at[pl.ds(base + (c + 1) * _CH, _CH)], o1)

        @pl.when(c + 3 < _NCH)
        def _():
            fire_gather(c + 3, rb1, g1)

        return c + 2

    lax.fori_loop(0, _NCH // 2, step_wrap, 0)
    wait_out(ob0, o0)
    wait_out(ob1, o1)


def _sc_lookup(x_flat, m1_flat, m2_flat, m3_flat, u):
    mesh = plsc.VectorSubcoreMesh(core_axis_name="c", subcore_axis_name="s")
    fn = functools.partial(
        pl.kernel,
        mesh=mesh,
        out_type=jax.ShapeDtypeStruct((TOK, D), jnp.float32),
        scratch_types=[
            pltpu.VMEM((_TPW,), jnp.int32),
            pltpu.VMEM((_TPW,), jnp.int32),
            pltpu.VMEM((_TPW,), jnp.int32),
            pltpu.VMEM((_TPW,), jnp.int32),
            pltpu.VMEM((_CH, _PK), jnp.uint32),
            pltpu.VMEM((_CH, _PK), jnp.uint32),
            pltpu.VMEM((_CH, D), jnp.float32),
            pltpu.VMEM((_CH, D), jnp.float32),
            pltpu.SemaphoreType.DMA,
            pltpu.SemaphoreType.DMA,
            pltpu.SemaphoreType.DMA,
            pltpu.SemaphoreType.DMA,
        ],
    )(_sc_body)
    return fn(x_flat, m1_flat, m2_flat, m3_flat, u)


def kernel(x, cl_mask, lm_mask, cl_prime_mask, weight,
           lora_A_cl, lora_B_cl, lora_A_lm, lora_B_lm,
           lora_A_cl_prime, lora_B_cl_prime):
    b, l = x.shape
    u = _make_table(weight, lora_A_cl, lora_B_cl, lora_A_lm, lora_B_lm,
                    lora_A_cl_prime, lora_B_cl_prime)
    out = _sc_lookup(x.reshape(TOK), cl_mask.reshape(TOK),
                     lm_mask.reshape(TOK), cl_prime_mask.reshape(TOK), u)
    return out.reshape(b, l, D)


# speedup vs baseline: 9.8269x; 1.0647x over previous
import functools

import numpy as np

import jax
import jax.numpy as jnp
from jax import lax
from jax.experimental import pallas as pl
from jax.experimental.pallas import tpu as pltpu
from jax.experimental.pallas import tpu_sc as plsc

V = 100000
D = 128
R = 128
TOK = 1024 * 200

_PK = 4 * D // 2


_VB = 1000


def _pack_section(o, i, s):
    su = lax.bitcast_convert_type(
        s.astype(jnp.bfloat16).astype(jnp.float32), jnp.uint32)
    au = su[:, 0:64] >> 16
    bu = su[:, 64:128] & np.uint32(0xFFFF0000)
    o[:, 64 * i:64 * i + 64] = au | bu


def _u_body(w, a1, b1, a2, b2, a3, b3, a01, a02, a03, o):
    t01 = 2.0 * jnp.dot(a01[...], b1[...], preferred_element_type=jnp.float32)
    t02 = 2.0 * jnp.dot(a02[...], b2[...], preferred_element_type=jnp.float32)
    t03 = 2.0 * jnp.dot(a03[...], b3[...], preferred_element_type=jnp.float32)
    _pack_section(o, 0, w[...] + (t01 + (t02 + t03)))
    _pack_section(o, 1, 2.0 * jnp.dot(a1[...], b1[...],
                                      preferred_element_type=jnp.float32) - t01)
    _pack_section(o, 2, 2.0 * jnp.dot(a2[...], b2[...],
                                      preferred_element_type=jnp.float32) - t02)
    _pack_section(o, 3, 2.0 * jnp.dot(a3[...], b3[...],
                                      preferred_element_type=jnp.float32) - t03)


def _make_table(w, a1, b1, a2, b2, a3, b3):
    grid = (V // _VB,)
    r_spec = pl.BlockSpec((_VB, R), lambda i: (i, 0))
    b_spec = pl.BlockSpec((R, D), lambda i: (0, 0))
    r0_spec = pl.BlockSpec((1, R), lambda i: (0, 0))
    o_spec = pl.BlockSpec((_VB, _PK), lambda i: (i, 0))
    return pl.pallas_call(
        _u_body,
        grid=grid,
        in_specs=[r_spec, r_spec, b_spec, r_spec, b_spec, r_spec, b_spec,
                  r0_spec, r0_spec, r0_spec],
        out_specs=o_spec,
        out_shape=jax.ShapeDtypeStruct((V, _PK), jnp.uint32),
    )(w, a1, b1, a2, b2, a3, b3, a1[0:1], a2[0:1], a3[0:1])


def _bcast_lane(vec, lane):
    idx = jnp.full((16, 1), lane, jnp.int32)
    dn = lax.GatherDimensionNumbers(
        offset_dims=(), collapsed_slice_dims=(0,), start_index_map=(0,))
    return lax.gather(vec, idx, dn, (1,),
                      mode=lax.GatherScatterMode.PROMISE_IN_BOUNDS)


_CH = 128
_NW = 32
_TPW = TOK // _NW
_NCH = _TPW // _CH

def _unpack2(v):
    a = lax.bitcast_convert_type(v << 16, jnp.float32)
    b = lax.bitcast_convert_type(v & np.uint32(0xFFFF0000), jnp.float32)
    return a, b


def _sc_body(x_hbm, m1_hbm, m2_hbm, m3_hbm, u_hbm, out_hbm,
             xb, p1, p2, p3, rb0, rb1, ob0, ob1,
             g0, g1, o0, o1):
    nc = 2
    wid = lax.axis_index("s") * nc + lax.axis_index("c")
    base = wid * _TPW

    pltpu.sync_copy(x_hbm.at[pl.ds(base, _TPW)], xb)
    pltpu.sync_copy(m1_hbm.at[pl.ds(base, _TPW)], p1)
    pltpu.sync_copy(m2_hbm.at[pl.ds(base, _TPW)], p2)
    pltpu.sync_copy(m3_hbm.at[pl.ds(base, _TPW)], p3)

    def idx_body(g, _):
        s = pl.ds(g * 16, 16)
        xv = xb[s]
        p1[s] = xv * p1[s]
        p2[s] = xv * p2[s]
        p3[s] = xv * p3[s]
        return 0

    lax.fori_loop(0, _TPW // 16, idx_body, 0)

    def fire_gather(c, rb, sem):
        return pltpu.async_copy(u_hbm.at[xb.at[pl.ds(c * _CH, _CH)]], rb, sem)

    def wait_gather(rb, sem):
        pltpu.make_async_copy(u_hbm.at[pl.ds(0, _CH)], rb, sem).wait()

    def wait_out(ob, sem):
        pltpu.make_async_copy(ob, out_hbm.at[pl.ds(0, _CH)], sem).wait()

    def combine(rb, ob, c):
        toff0 = c * _CH

        def grp_body(q, _):
            gsl = pl.ds(toff0 + q * 16, 16)
            one = jnp.full((16,), 1, jnp.int32)
            iv1 = jnp.minimum(p1[gsl], one).astype(jnp.float32)
            iv2 = jnp.minimum(p2[gsl], one).astype(jnp.float32)
            iv3 = jnp.minimum(p3[gsl], one).astype(jnp.float32)
            for rr in range(16):
                r = q * 16 + rr
                b1 = _bcast_lane(iv1, rr)
                b2 = _bcast_lane(iv2, rr)
                b3 = _bcast_lane(iv3, rr)
                for g2 in range(4):
                    wv = rb[r, pl.ds(g2 * 16, 16)]
                    t1 = rb[r, pl.ds(64 + g2 * 16, 16)]
                    t2 = rb[r, pl.ds(128 + g2 * 16, 16)]
                    t3 = rb[r, pl.ds(192 + g2 * 16, 16)]
                    wa, wb = _unpack2(wv)
                    t1a, t1b = _unpack2(t1)
                    t2a, t2b = _unpack2(t2)
                    t3a, t3b = _unpack2(t3)
                    ob[r, pl.ds(16 * g2, 16)] = (
                        (wa + b1 * t1a) + (b2 * t2a + b3 * t3a))
                    ob[r, pl.ds(64 + 16 * g2, 16)] = (
                        (wb + b1 * t1b) + (b2 * t2b + b3 * t3b))
            return 0

        lax.fori_loop(0, _CH // 16, grp_body, 0)

    fire_gather(0, rb0, g0)
    fire_gather(1, rb1, g1)

    def step_wrap(j, c):
        wait_gather(rb0, g0)

        @pl.when(j > 0)
        def _():
            wait_out(ob0, o0)

        combine(rb0, ob0, c)
        pltpu.async_copy(ob0, out_hbm.at[pl.ds(base + c * _CH, _CH)], o0)

        @pl.when(c + 2 < _NCH)
        def _():
            fire_gather(c + 2, rb0, g0)

        wait_gather(rb1, g1)

        @pl.when(j > 0)
        def _():
            wait_out(ob1, o1)

        combine(rb1, ob1, c + 1)
        pltpu.async_copy(ob1, out_hbm.at[pl.ds(base + (c + 1) * _CH, _CH)], o1)

        @pl.when(c + 3 < _NCH)
        def _():
            fire_gather(c + 3, rb1, g1)

        return c + 2

    lax.fori_loop(0, _NCH // 2, step_wrap, 0)
    wait_out(ob0, o0)
    wait_out(ob1, o1)


def _sc_lookup(x_flat, m1_flat, m2_flat, m3_flat, u):
    mesh = plsc.VectorSubcoreMesh(core_axis_name="c", subcore_axis_name="s")
    fn = functools.partial(
        pl.kernel,
        mesh=mesh,
        out_type=jax.ShapeDtypeStruct((TOK, D), jnp.float32),
        scratch_types=[
            pltpu.VMEM((_TPW,), jnp.int32),
            pltpu.VMEM((_TPW,), jnp.int32),
            pltpu.VMEM((_TPW,), jnp.int32),
            pltpu.VMEM((_TPW,), jnp.int32),
            pltpu.VMEM((_CH, _PK), jnp.uint32),
            pltpu.VMEM((_CH, _PK), jnp.uint32),
            pltpu.VMEM((_CH, D), jnp.float32),
            pltpu.VMEM((_CH, D), jnp.float32),
            pltpu.SemaphoreType.DMA,
            pltpu.SemaphoreType.DMA,
            pltpu.SemaphoreType.DMA,
            pltpu.SemaphoreType.DMA,
        ],
    )(_sc_body)
    return fn(x_flat, m1_flat, m2_flat, m3_flat, u)


def kernel(x, cl_mask, lm_mask, cl_prime_mask, weight,
           lora_A_cl, lora_B_cl, lora_A_lm, lora_B_lm,
           lora_A_cl_prime, lora_B_cl_prime):
    b, l = x.shape
    u = _make_table(weight, lora_A_cl, lora_B_cl, lora_A_lm, lora_B_lm,
                    lora_A_cl_prime, lora_B_cl_prime)
    out = _sc_lookup(x.reshape(TOK), cl_mask.reshape(TOK),
                     lm_mask.reshape(TOK), cl_prime_mask.reshape(TOK), u)
    return out.reshape(b, l, D)
